# baseline (speedup 1.0000x reference)
import functools

import numpy as np
import jax
import jax.numpy as jnp
from jax import lax
from jax.experimental import pallas as pl
from jax.experimental.pallas import tpu as pltpu
from jax.experimental.pallas import tpu_sc as plsc

B = 2
NQ = 4096
NKV = 4096
C = 384
NH = 6
HD = C // NH
SR = 2
NS = 1024
SCALE = HD ** -0.5
INV4 = 1.0 / (4.0 + 1e-6)
EPS = 1e-5
N_INIT = 128 * 128
CHUNK = 128


def _build_perm():
    kh = np.arange(2).reshape(2, 1, 1, 1, 1, 1)
    kw = np.arange(2).reshape(1, 2, 1, 1, 1, 1)
    a = np.arange(2).reshape(1, 1, 2, 1, 1, 1)
    b = np.arange(2).reshape(1, 1, 1, 2, 1, 1)
    r = np.arange(32).reshape(1, 1, 1, 1, 32, 1)
    c = np.arange(32).reshape(1, 1, 1, 1, 1, 32)
    i = 4 * r + 2 * kh + a
    j = 4 * c + 2 * kw + b
    return jnp.asarray((i * 128 + j).reshape(-1), jnp.int32)


_PERM = _build_perm()


def _sc_gather(table, idx2d):
    info = plsc.get_sparse_core_info()
    nw = info.num_cores * info.num_subcores
    per_w = N_INIT // nw
    nch = per_w // CHUNK
    mesh = plsc.VectorSubcoreMesh(core_axis_name="c", subcore_axis_name="s")

    @functools.partial(
        pl.kernel,
        mesh=mesh,
        out_type=jax.ShapeDtypeStruct((N_INIT, C), jnp.float32),
        scratch_types=[
            pltpu.VMEM((nch, CHUNK), jnp.int32),
            pltpu.VMEM((CHUNK, C), jnp.float32),
            pltpu.VMEM((CHUNK, C), jnp.float32),
            pltpu.SemaphoreType.DMA,
            pltpu.SemaphoreType.DMA,
        ],
    )
    def gk(table_hbm, idx_hbm, out_hbm, idx_v, buf0, buf1, sem0, sem1):
        wid = lax.axis_index("s") * info.num_cores + lax.axis_index("c")
        base = wid * per_w
        pltpu.sync_copy(idx_hbm.at[pl.ds(wid * nch, nch)], idx_v)
        bufs, sems = (buf0, buf1), (sem0, sem1)
        handles = [None, None]
        handles[0] = pltpu.async_copy(table_hbm.at[idx_v.at[0]], buf0, sem0)
        for ci in range(nch):
            t = ci % 2
            if ci + 1 < nch:
                handles[1 - t] = pltpu.async_copy(
                    table_hbm.at[idx_v.at[ci + 1]], bufs[1 - t], sems[1 - t])
            handles[t].wait()
            pltpu.sync_copy(bufs[t],
                            out_hbm.at[pl.ds(base + ci * CHUNK, CHUNK)])

    return gk(table, idx2d)


def _kv_path(g4, w2s, srb, lng, lnb, wkv):
    BCC = 256

    def body(g_ref, w2_ref, srb_ref, lng_ref, lnb_ref, wkv_ref, out_ref):
        acc = jnp.broadcast_to(srb_ref[...], (BCC, C)).astype(jnp.float32)
        for k in range(4):
            mk = (g_ref[k, 0] + g_ref[k, 1] + g_ref[k, 2] + g_ref[k, 3])
            acc = acc + jnp.dot(mk.astype(jnp.bfloat16), w2_ref[k],
                                preferred_element_type=jnp.float32)
        mu = jnp.mean(acc, axis=-1, keepdims=True)
        xc = acc - mu
        var = jnp.mean(xc * xc, axis=-1, keepdims=True)
        ln = xc * lax.rsqrt(var + EPS) * lng_ref[...] + lnb_ref[...]
        out_ref[...] = jnp.dot(ln.astype(jnp.bfloat16), wkv_ref[...],
                               preferred_element_type=jnp.float32
                               ).astype(jnp.bfloat16)

    return pl.pallas_call(
        body,
        grid=(NS // BCC,),
        in_specs=[
            pl.BlockSpec((4, 4, BCC, C), lambda i: (0, 0, i, 0)),
            pl.BlockSpec((4, C, C), lambda i: (0, 0, 0)),
            pl.BlockSpec((1, C), lambda i: (0, 0)),
            pl.BlockSpec((1, C), lambda i: (0, 0)),
            pl.BlockSpec((1, C), lambda i: (0, 0)),
            pl.BlockSpec((C, 2 * C), lambda i: (0, 0)),
        ],
        out_specs=pl.BlockSpec((BCC, 2 * C), lambda i: (i, 0)),
        out_shape=jax.ShapeDtypeStruct((NS, 2 * C), jnp.bfloat16),
    )(g4, w2s, srb, lng, lnb, wkv)


def _attention(q_x, wqs, kv, wp, bp):
    BQ = 512

    def body(qx_ref, wq_ref, kv_ref, wp_ref, bp_ref, out_ref):
        q = jnp.dot(qx_ref[...].astype(jnp.bfloat16), wq_ref[...],
                    preferred_element_type=jnp.float32).astype(jnp.bfloat16)
        outs = []
        for h in range(NH):
            qh = q[:, h * HD:(h + 1) * HD]
            kh = kv_ref[:, h * HD:(h + 1) * HD]
            vh = kv_ref[:, C + h * HD:C + (h + 1) * HD]
            s = lax.dot_general(qh, kh, (((1,), (1,)), ((), ())),
                                preferred_element_type=jnp.float32)
            m = jnp.max(s, axis=-1, keepdims=True)
            p = jnp.exp(s - m).astype(jnp.bfloat16)
            d = jnp.sum(p, axis=-1, keepdims=True, dtype=jnp.float32)
            outs.append(jnp.dot(p, vh, preferred_element_type=jnp.float32) / d)
        acc = jnp.concatenate(outs, axis=-1)
        out_ref[...] = jnp.dot(acc.astype(jnp.bfloat16), wp_ref[...],
                               preferred_element_type=jnp.float32) + bp_ref[...]

    return pl.pallas_call(
        body,
        grid=(NQ // BQ,),
        in_specs=[
            pl.BlockSpec((BQ, C), lambda i: (i, 0)),
            pl.BlockSpec((C, C), lambda i: (0, 0)),
            pl.BlockSpec((NS, 2 * C), lambda i: (0, 0)),
            pl.BlockSpec((C, C), lambda i: (0, 0)),
            pl.BlockSpec((1, C), lambda i: (0, 0)),
        ],
        out_specs=pl.BlockSpec((BQ, C), lambda i: (i, 0)),
        out_shape=jax.ShapeDtypeStruct((NQ, C), jnp.float32),
    )(q_x, wqs, kv, wp, bp)


def kernel(q_x, kv_x, idx_token, Wq, Wkv, sr_w, sr_b, ln_g, ln_b, Wp, bp):
    idx32 = idx_token.astype(jnp.int32)
    w2s = (jnp.transpose(sr_w, (2, 3, 1, 0)).reshape(4, C, C)
           * INV4).astype(jnp.bfloat16)
    wqs = (Wq * SCALE).astype(jnp.bfloat16)
    wkv16 = Wkv.astype(jnp.bfloat16)
    wp16 = Wp.astype(jnp.bfloat16)
    srb = sr_b.reshape(1, C)
    lng = ln_g.reshape(1, C)
    lnb = ln_b.reshape(1, C)
    bp2 = bp.reshape(1, C)
    outs = []
    for b in range(B):
        idx2d = idx32[b, _PERM].reshape(N_INIT // CHUNK, CHUNK)
        g = _sc_gather(kv_x[b], idx2d)
        g4 = g.reshape(4, 4, NS, C)
        kv = _kv_path(g4, w2s, srb, lng, lnb, wkv16)
        outs.append(_attention(q_x[b], wqs, kv, wp16, bp2))
    return jnp.stack(outs)


# speedup vs baseline: 1.0997x; 1.0997x over previous
import functools

import numpy as np
import jax
import jax.numpy as jnp
from jax import lax
from jax.experimental import pallas as pl
from jax.experimental.pallas import tpu as pltpu
from jax.experimental.pallas import tpu_sc as plsc

B = 2
NQ = 4096
NKV = 4096
C = 384
NH = 6
HD = C // NH
SR = 2
NS = 1024
SCALE = HD ** -0.5
INV4 = 1.0 / (4.0 + 1e-6)
EPS = 1e-5
N_INIT = 128 * 128
CHUNK = 128


def _build_perm():
    kh = np.arange(2).reshape(2, 1, 1, 1, 1, 1)
    kw = np.arange(2).reshape(1, 2, 1, 1, 1, 1)
    a = np.arange(2).reshape(1, 1, 2, 1, 1, 1)
    b = np.arange(2).reshape(1, 1, 1, 2, 1, 1)
    r = np.arange(32).reshape(1, 1, 1, 1, 32, 1)
    c = np.arange(32).reshape(1, 1, 1, 1, 1, 32)
    i = 4 * r + 2 * kh + a
    j = 4 * c + 2 * kw + b
    return jnp.asarray((i * 128 + j).reshape(-1), jnp.int32)


_PERM = _build_perm()


def _sc_gather(table, idx2d):
    info = plsc.get_sparse_core_info()
    nw = info.num_cores * info.num_subcores
    per_w = N_INIT // nw
    nch = per_w // CHUNK
    mesh = plsc.VectorSubcoreMesh(core_axis_name="c", subcore_axis_name="s")

    @functools.partial(
        pl.kernel,
        mesh=mesh,
        out_type=jax.ShapeDtypeStruct((N_INIT, C), jnp.float32),
        scratch_types=[
            pltpu.VMEM((nch, CHUNK), jnp.int32),
            pltpu.VMEM((CHUNK, C), jnp.float32),
            pltpu.VMEM((CHUNK, C), jnp.float32),
            pltpu.SemaphoreType.DMA,
            pltpu.SemaphoreType.DMA,
        ],
    )
    def gk(table_hbm, idx_hbm, out_hbm, idx_v, buf0, buf1, sem0, sem1):
        wid = lax.axis_index("s") * info.num_cores + lax.axis_index("c")
        base = wid * per_w
        pltpu.sync_copy(idx_hbm.at[pl.ds(wid * nch, nch)], idx_v)
        bufs, sems = (buf0, buf1), (sem0, sem1)
        handles = [None, None]
        handles[0] = pltpu.async_copy(table_hbm.at[idx_v.at[0]], buf0, sem0)
        for ci in range(nch):
            t = ci % 2
            if ci + 1 < nch:
                handles[1 - t] = pltpu.async_copy(
                    table_hbm.at[idx_v.at[ci + 1]], bufs[1 - t], sems[1 - t])
            handles[t].wait()
            pltpu.sync_copy(bufs[t],
                            out_hbm.at[pl.ds(base + ci * CHUNK, CHUNK)])

    return gk(table, idx2d)


def _kv_path(g4, w2s, srb, lng, lnb, wkv):
    BCC = 256

    def body(g_ref, w2_ref, srb_ref, lng_ref, lnb_ref, wkv_ref, out_ref):
        acc = jnp.broadcast_to(srb_ref[...], (BCC, C)).astype(jnp.float32)
        for k in range(4):
            mk = (g_ref[k, 0] + g_ref[k, 1] + g_ref[k, 2] + g_ref[k, 3])
            acc = acc + jnp.dot(mk.astype(jnp.bfloat16), w2_ref[k],
                                preferred_element_type=jnp.float32)
        mu = jnp.mean(acc, axis=-1, keepdims=True)
        xc = acc - mu
        var = jnp.mean(xc * xc, axis=-1, keepdims=True)
        ln = xc * lax.rsqrt(var + EPS) * lng_ref[...] + lnb_ref[...]
        out_ref[...] = jnp.dot(ln.astype(jnp.bfloat16), wkv_ref[...],
                               preferred_element_type=jnp.float32
                               ).astype(jnp.bfloat16)

    return pl.pallas_call(
        body,
        grid=(NS // BCC,),
        in_specs=[
            pl.BlockSpec((4, 4, BCC, C), lambda i: (0, 0, i, 0)),
            pl.BlockSpec((4, C, C), lambda i: (0, 0, 0)),
            pl.BlockSpec((1, C), lambda i: (0, 0)),
            pl.BlockSpec((1, C), lambda i: (0, 0)),
            pl.BlockSpec((1, C), lambda i: (0, 0)),
            pl.BlockSpec((C, 2 * C), lambda i: (0, 0)),
        ],
        out_specs=pl.BlockSpec((BCC, 2 * C), lambda i: (i, 0)),
        out_shape=jax.ShapeDtypeStruct((NS, 2 * C), jnp.bfloat16),
    )(g4, w2s, srb, lng, lnb, wkv)


def _attention(q_x, wqs, kv, wp, bp, out_prev, bsel):
    BQ = 512

    def body(qx_ref, wq_ref, kv_ref, wp_ref, bp_ref, _prev_ref, out_ref):
        q = jnp.dot(qx_ref[0].astype(jnp.bfloat16), wq_ref[...],
                    preferred_element_type=jnp.float32).astype(jnp.bfloat16)
        outs = []
        for h in range(NH):
            qh = q[:, h * HD:(h + 1) * HD]
            kh = kv_ref[:, h * HD:(h + 1) * HD]
            vh = kv_ref[:, C + h * HD:C + (h + 1) * HD]
            s = lax.dot_general(qh, kh, (((1,), (1,)), ((), ())),
                                preferred_element_type=jnp.float32)
            m = jnp.max(s, axis=-1, keepdims=True)
            p = jnp.exp(s - m).astype(jnp.bfloat16)
            d = jnp.sum(p, axis=-1, keepdims=True, dtype=jnp.float32)
            outs.append(jnp.dot(p, vh, preferred_element_type=jnp.float32) / d)
        acc = jnp.concatenate(outs, axis=-1)
        out_ref[0] = jnp.dot(acc.astype(jnp.bfloat16), wp_ref[...],
                             preferred_element_type=jnp.float32) + bp_ref[...]

    return pl.pallas_call(
        body,
        grid=(NQ // BQ,),
        in_specs=[
            pl.BlockSpec((1, BQ, C), lambda i: (bsel, i, 0)),
            pl.BlockSpec((C, C), lambda i: (0, 0)),
            pl.BlockSpec((NS, 2 * C), lambda i: (0, 0)),
            pl.BlockSpec((C, C), lambda i: (0, 0)),
            pl.BlockSpec((1, C), lambda i: (0, 0)),
            pl.BlockSpec((1, BQ, C), lambda i: (bsel, i, 0)),
        ],
        out_specs=pl.BlockSpec((1, BQ, C), lambda i: (bsel, i, 0)),
        out_shape=jax.ShapeDtypeStruct((B, NQ, C), jnp.float32),
        input_output_aliases={5: 0},
    )(q_x, wqs, kv, wp, bp, out_prev)


def kernel(q_x, kv_x, idx_token, Wq, Wkv, sr_w, sr_b, ln_g, ln_b, Wp, bp):
    idx32 = idx_token.astype(jnp.int32)
    w2s = (jnp.transpose(sr_w, (2, 3, 1, 0)).reshape(4, C, C)
           * INV4).astype(jnp.bfloat16)
    wqs = (Wq * SCALE).astype(jnp.bfloat16)
    wkv16 = Wkv.astype(jnp.bfloat16)
    wp16 = Wp.astype(jnp.bfloat16)
    srb = sr_b.reshape(1, C)
    lng = ln_g.reshape(1, C)
    lnb = ln_b.reshape(1, C)
    bp2 = bp.reshape(1, C)
    table = kv_x.reshape(B * NKV, C)
    out = jnp.zeros((B, NQ, C), jnp.float32)
    for b in range(B):
        idx2d = (idx32[b, _PERM] + b * NKV).reshape(N_INIT // CHUNK, CHUNK)
        g = _sc_gather(table, idx2d)
        g4 = g.reshape(4, 4, NS, C)
        kv = _kv_path(g4, w2s, srb, lng, lnb, wkv16)
        out = _attention(q_x, wqs, kv, wp16, bp2, out, b)
    return out


# speedup vs baseline: 1.1108x; 1.0101x over previous
import functools

import numpy as np
import jax
import jax.numpy as jnp
from jax import lax
from jax.experimental import pallas as pl
from jax.experimental.pallas import tpu as pltpu
from jax.experimental.pallas import tpu_sc as plsc

B = 2
NQ = 4096
NKV = 4096
C = 384
NH = 6
HD = C // NH
SR = 2
NS = 1024
SCALE = HD ** -0.5
INV4 = 1.0 / (4.0 + 1e-6)
EPS = 1e-5
N_INIT = 128 * 128
CHUNK = 128


def _build_perm():
    kh = np.arange(2).reshape(2, 1, 1, 1, 1, 1)
    kw = np.arange(2).reshape(1, 2, 1, 1, 1, 1)
    a = np.arange(2).reshape(1, 1, 2, 1, 1, 1)
    b = np.arange(2).reshape(1, 1, 1, 2, 1, 1)
    r = np.arange(32).reshape(1, 1, 1, 1, 32, 1)
    c = np.arange(32).reshape(1, 1, 1, 1, 1, 32)
    i = 4 * r + 2 * kh + a
    j = 4 * c + 2 * kw + b
    return jnp.asarray((i * 128 + j).reshape(-1), jnp.int32)


_PERM = _build_perm()


def _sc_gather(table, idx2d):
    info = plsc.get_sparse_core_info()
    nw = info.num_cores * info.num_subcores
    per_w = N_INIT // nw
    nch = per_w // CHUNK
    mesh = plsc.VectorSubcoreMesh(core_axis_name="c", subcore_axis_name="s")

    @functools.partial(
        pl.kernel,
        mesh=mesh,
        out_type=jax.ShapeDtypeStruct((N_INIT, C), jnp.float32),
        scratch_types=[
            pltpu.VMEM((nch, CHUNK), jnp.int32),
            pltpu.VMEM((CHUNK, C), jnp.float32),
            pltpu.VMEM((CHUNK, C), jnp.float32),
            pltpu.SemaphoreType.DMA,
            pltpu.SemaphoreType.DMA,
        ],
    )
    def gk(table_hbm, idx_hbm, out_hbm, idx_v, buf0, buf1, sem0, sem1):
        wid = lax.axis_index("s") * info.num_cores + lax.axis_index("c")
        base = wid * per_w
        pltpu.sync_copy(idx_hbm.at[pl.ds(wid * nch, nch)], idx_v)
        bufs, sems = (buf0, buf1), (sem0, sem1)
        handles = [None, None]
        handles[0] = pltpu.async_copy(table_hbm.at[idx_v.at[0]], buf0, sem0)
        for ci in range(nch):
            t = ci % 2
            if ci + 1 < nch:
                handles[1 - t] = pltpu.async_copy(
                    table_hbm.at[idx_v.at[ci + 1]], bufs[1 - t], sems[1 - t])
            handles[t].wait()
            pltpu.sync_copy(bufs[t],
                            out_hbm.at[pl.ds(base + ci * CHUNK, CHUNK)])

    return gk(table, idx2d)


def _kv_path(g4, w2s, srb, lng, lnb, wkv):
    BCC = 256

    def body(g_ref, w2_ref, srb_ref, lng_ref, lnb_ref, wkv_ref, out_ref):
        acc = jnp.broadcast_to(srb_ref[...], (BCC, C)).astype(jnp.float32)
        for k in range(4):
            mk = (g_ref[k, 0] + g_ref[k, 1] + g_ref[k, 2] + g_ref[k, 3])
            acc = acc + jnp.dot(mk.astype(jnp.bfloat16), w2_ref[k],
                                preferred_element_type=jnp.float32)
        mu = jnp.mean(acc, axis=-1, keepdims=True)
        xc = acc - mu
        var = jnp.mean(xc * xc, axis=-1, keepdims=True)
        ln = xc * lax.rsqrt(var + EPS) * lng_ref[...] + lnb_ref[...]
        out_ref[...] = jnp.dot(ln.astype(jnp.bfloat16), wkv_ref[...],
                               preferred_element_type=jnp.float32
                               ).astype(jnp.bfloat16)

    return pl.pallas_call(
        body,
        grid=(NS // BCC,),
        in_specs=[
            pl.BlockSpec((4, 4, BCC, C), lambda i: (0, 0, i, 0)),
            pl.BlockSpec((4, C, C), lambda i: (0, 0, 0)),
            pl.BlockSpec((1, C), lambda i: (0, 0)),
            pl.BlockSpec((1, C), lambda i: (0, 0)),
            pl.BlockSpec((1, C), lambda i: (0, 0)),
            pl.BlockSpec((C, 2 * C), lambda i: (0, 0)),
        ],
        out_specs=pl.BlockSpec((BCC, 2 * C), lambda i: (i, 0)),
        out_shape=jax.ShapeDtypeStruct((NS, 2 * C), jnp.bfloat16),
    )(g4, w2s, srb, lng, lnb, wkv)


def _attention(q_x, wqs, kv, wp, bp, out_prev, bsel):
    BQ = 512

    def body(qx_ref, wq_ref, kv_ref, wp_ref, bp_ref, _prev_ref, out_ref):
        q = jnp.dot(qx_ref[0].astype(jnp.bfloat16), wq_ref[...],
                    preferred_element_type=jnp.float32).astype(jnp.bfloat16)
        outs = []
        for h in range(NH):
            qh = q[:, h * HD:(h + 1) * HD]
            kh = kv_ref[:, h * HD:(h + 1) * HD]
            vh = kv_ref[:, C + h * HD:C + (h + 1) * HD]
            s = lax.dot_general(qh, kh, (((1,), (1,)), ((), ())),
                                preferred_element_type=jnp.float32)
            m = jnp.max(s, axis=-1, keepdims=True)
            p = jnp.exp(s - m).astype(jnp.bfloat16)
            d = jnp.sum(p, axis=-1, keepdims=True, dtype=jnp.float32)
            outs.append(jnp.dot(p, vh, preferred_element_type=jnp.float32) / d)
        acc = jnp.concatenate(outs, axis=-1)
        out_ref[0] = jnp.dot(acc.astype(jnp.bfloat16), wp_ref[...],
                             preferred_element_type=jnp.float32) + bp_ref[...]

    return pl.pallas_call(
        body,
        grid=(NQ // BQ,),
        in_specs=[
            pl.BlockSpec((1, BQ, C), lambda i: (bsel, i, 0)),
            pl.BlockSpec((C, C), lambda i: (0, 0)),
            pl.BlockSpec((NS, 2 * C), lambda i: (0, 0)),
            pl.BlockSpec((C, C), lambda i: (0, 0)),
            pl.BlockSpec((1, C), lambda i: (0, 0)),
            pl.BlockSpec((1, BQ, C), lambda i: (bsel, i, 0)),
        ],
        out_specs=pl.BlockSpec((1, BQ, C), lambda i: (bsel, i, 0)),
        out_shape=jax.ShapeDtypeStruct((B, NQ, C), jnp.float32),
        input_output_aliases={5: 0},
    )(q_x, wqs, kv, wp, bp, out_prev)


def kernel(q_x, kv_x, idx_token, Wq, Wkv, sr_w, sr_b, ln_g, ln_b, Wp, bp):
    idx32 = idx_token.astype(jnp.int32)
    w2s = (jnp.transpose(sr_w, (2, 3, 1, 0)).reshape(4, C, C)
           * INV4).astype(jnp.bfloat16)
    wqs = (Wq * SCALE).astype(jnp.bfloat16)
    wkv16 = Wkv.astype(jnp.bfloat16)
    wp16 = Wp.astype(jnp.bfloat16)
    srb = sr_b.reshape(1, C)
    lng = ln_g.reshape(1, C)
    lnb = ln_b.reshape(1, C)
    bp2 = bp.reshape(1, C)
    table = kv_x.reshape(B * NKV, C)
    out = jnp.zeros((B, NQ, C), jnp.float32)
    for b in range(B):
        idx2d = (idx32[b, _PERM] + b * NKV).reshape(N_INIT // CHUNK, CHUNK)
        g = _sc_gather(table, idx2d)
        g4 = g.reshape(4, 4, NS, C)
        srb_b = srb if b == 0 else srb + out[0, :1, :] * 0.0
        kv = _kv_path(g4, w2s, srb_b, lng, lnb, wkv16)
        out = _attention(q_x, wqs, kv, wp16, bp2, out, b)
    return out


# speedup vs baseline: 1.1878x; 1.0693x over previous
import functools

import numpy as np
import jax
import jax.numpy as jnp
from jax import lax
from jax.experimental import pallas as pl
from jax.experimental.pallas import tpu as pltpu
from jax.experimental.pallas import tpu_sc as plsc

B = 2
NQ = 4096
NKV = 4096
C = 384
NH = 6
HD = C // NH
SR = 2
NS = 1024
SCALE = HD ** -0.5
INV4 = 1.0 / (4.0 + 1e-6)
EPS = 1e-5
N_INIT = 128 * 128
CHUNK = 128


def _build_perm():
    kh = np.arange(2).reshape(2, 1, 1, 1, 1, 1)
    kw = np.arange(2).reshape(1, 2, 1, 1, 1, 1)
    a = np.arange(2).reshape(1, 1, 2, 1, 1, 1)
    b = np.arange(2).reshape(1, 1, 1, 2, 1, 1)
    r = np.arange(32).reshape(1, 1, 1, 1, 32, 1)
    c = np.arange(32).reshape(1, 1, 1, 1, 1, 32)
    i = 4 * r + 2 * kh + a
    j = 4 * c + 2 * kw + b
    return jnp.asarray((i * 128 + j).reshape(-1), jnp.int32)


_PERM = _build_perm()


def _sc_gather(table, idx2d):
    info = plsc.get_sparse_core_info()
    nw = info.num_cores * info.num_subcores
    per_w = N_INIT // nw
    nch = per_w // CHUNK
    mesh = plsc.VectorSubcoreMesh(core_axis_name="c", subcore_axis_name="s")

    @functools.partial(
        pl.kernel,
        mesh=mesh,
        out_type=jax.ShapeDtypeStruct((N_INIT, C), jnp.float32),
        scratch_types=[
            pltpu.VMEM((nch, CHUNK), jnp.int32),
            pltpu.VMEM((CHUNK, C), jnp.float32),
            pltpu.VMEM((CHUNK, C), jnp.float32),
            pltpu.SemaphoreType.DMA,
            pltpu.SemaphoreType.DMA,
        ],
    )
    def gk(table_hbm, idx_hbm, out_hbm, idx_v, buf0, buf1, sem0, sem1):
        wid = lax.axis_index("s") * info.num_cores + lax.axis_index("c")
        base = wid * per_w
        pltpu.sync_copy(idx_hbm.at[pl.ds(wid * nch, nch)], idx_v)
        bufs, sems = (buf0, buf1), (sem0, sem1)
        handles = [None, None]
        handles[0] = pltpu.async_copy(table_hbm.at[idx_v.at[0]], buf0, sem0)
        for ci in range(nch):
            t = ci % 2
            if ci + 1 < nch:
                handles[1 - t] = pltpu.async_copy(
                    table_hbm.at[idx_v.at[ci + 1]], bufs[1 - t], sems[1 - t])
            handles[t].wait()
            pltpu.sync_copy(bufs[t],
                            out_hbm.at[pl.ds(base + ci * CHUNK, CHUNK)])

    return gk(table, idx2d)


def _kv_path(g4, w2s, srb, lng, lnb, wkv):
    BCC = 256

    def body(g_ref, w2_ref, srb_ref, lng_ref, lnb_ref, wkv_ref, out_ref):
        acc = jnp.broadcast_to(srb_ref[...], (BCC, C)).astype(jnp.float32)
        for k in range(4):
            mk = (g_ref[k, 0] + g_ref[k, 1] + g_ref[k, 2] + g_ref[k, 3])
            acc = acc + jnp.dot(mk.astype(jnp.bfloat16), w2_ref[k],
                                preferred_element_type=jnp.float32)
        mu = jnp.mean(acc, axis=-1, keepdims=True)
        xc = acc - mu
        var = jnp.mean(xc * xc, axis=-1, keepdims=True)
        ln = xc * lax.rsqrt(var + EPS) * lng_ref[...] + lnb_ref[...]
        out_ref[...] = jnp.dot(ln.astype(jnp.bfloat16), wkv_ref[...],
                               preferred_element_type=jnp.float32
                               ).astype(jnp.bfloat16)

    return pl.pallas_call(
        body,
        grid=(NS // BCC,),
        in_specs=[
            pl.BlockSpec((4, 4, BCC, C), lambda i: (0, 0, i, 0)),
            pl.BlockSpec((4, C, C), lambda i: (0, 0, 0)),
            pl.BlockSpec((1, C), lambda i: (0, 0)),
            pl.BlockSpec((1, C), lambda i: (0, 0)),
            pl.BlockSpec((1, C), lambda i: (0, 0)),
            pl.BlockSpec((C, 2 * C), lambda i: (0, 0)),
        ],
        out_specs=pl.BlockSpec((BCC, 2 * C), lambda i: (i, 0)),
        out_shape=jax.ShapeDtypeStruct((NS, 2 * C), jnp.bfloat16),
    )(g4, w2s, srb, lng, lnb, wkv)


def _attention(q_x, wqs, kv, wp, bp, out_prev, bsel):
    BQ = 512

    def body(qx_ref, wq_ref, kv_ref, wp_ref, bp_ref, *rest):
        out_ref = rest[-1]
        q = jnp.dot(qx_ref[0].astype(jnp.bfloat16), wq_ref[...],
                    preferred_element_type=jnp.float32).astype(jnp.bfloat16)
        outs = []
        for h in range(NH):
            qh = q[:, h * HD:(h + 1) * HD]
            kh = kv_ref[:, h * HD:(h + 1) * HD]
            vh = kv_ref[:, C + h * HD:C + (h + 1) * HD]
            s = lax.dot_general(qh, kh, (((1,), (1,)), ((), ())),
                                preferred_element_type=jnp.float32)
            m = jnp.max(s, axis=-1, keepdims=True)
            p = jnp.exp(s - m).astype(jnp.bfloat16)
            d = jnp.sum(p, axis=-1, keepdims=True, dtype=jnp.float32)
            outs.append(jnp.dot(p, vh, preferred_element_type=jnp.float32) / d)
        acc = jnp.concatenate(outs, axis=-1)
        out_ref[0] = jnp.dot(acc.astype(jnp.bfloat16), wp_ref[...],
                             preferred_element_type=jnp.float32) + bp_ref[...]

    in_specs = [
        pl.BlockSpec((1, BQ, C), lambda i: (bsel, i, 0)),
        pl.BlockSpec((C, C), lambda i: (0, 0)),
        pl.BlockSpec((NS, 2 * C), lambda i: (0, 0)),
        pl.BlockSpec((C, C), lambda i: (0, 0)),
        pl.BlockSpec((1, C), lambda i: (0, 0)),
    ]
    args = [q_x, wqs, kv, wp, bp]
    aliases = {}
    if out_prev is not None:
        in_specs.append(pl.BlockSpec((1, BQ, C), lambda i: (bsel, i, 0)))
        args.append(out_prev)
        aliases = {5: 0}
    return pl.pallas_call(
        body,
        grid=(NQ // BQ,),
        in_specs=in_specs,
        out_specs=pl.BlockSpec((1, BQ, C), lambda i: (bsel, i, 0)),
        out_shape=jax.ShapeDtypeStruct((B, NQ, C), jnp.float32),
        input_output_aliases=aliases,
    )(*args)


def kernel(q_x, kv_x, idx_token, Wq, Wkv, sr_w, sr_b, ln_g, ln_b, Wp, bp):
    idx32 = idx_token.astype(jnp.int32)
    w2s = (jnp.transpose(sr_w, (2, 3, 1, 0)).reshape(4, C, C)
           * INV4).astype(jnp.bfloat16)
    wqs = (Wq * SCALE).astype(jnp.bfloat16)
    wkv16 = Wkv.astype(jnp.bfloat16)
    wp16 = Wp.astype(jnp.bfloat16)
    srb = sr_b.reshape(1, C)
    lng = ln_g.reshape(1, C)
    lnb = ln_b.reshape(1, C)
    bp2 = bp.reshape(1, C)
    table = kv_x.reshape(B * NKV, C)
    out = None
    for b in range(B):
        idx2d = (idx32[b, _PERM] + b * NKV).reshape(N_INIT // CHUNK, CHUNK)
        g = _sc_gather(table, idx2d)
        g4 = g.reshape(4, 4, NS, C)
        srb_b = srb if b == 0 else srb + out[0, :1, :] * 0.0
        kv = _kv_path(g4, w2s, srb_b, lng, lnb, wkv16)
        out = _attention(q_x, wqs, kv, wp16, bp2, out, b)
    return out


# speedup vs baseline: 1.2370x; 1.0414x over previous
import functools

import numpy as np
import jax
import jax.numpy as jnp
from jax import lax
from jax.experimental import pallas as pl
from jax.experimental.pallas import tpu as pltpu
from jax.experimental.pallas import tpu_sc as plsc

B = 2
NQ = 4096
NKV = 4096
C = 384
NH = 6
HD = C // NH
SR = 2
NS = 1024
SCALE = HD ** -0.5
VB = 128
KC = C + NH * VB
INV4 = 1.0 / (4.0 + 1e-6)
EPS = 1e-5
N_INIT = 128 * 128
CHUNK = 128


def _build_perm():
    kh = np.arange(2).reshape(2, 1, 1, 1, 1, 1)
    kw = np.arange(2).reshape(1, 2, 1, 1, 1, 1)
    a = np.arange(2).reshape(1, 1, 2, 1, 1, 1)
    b = np.arange(2).reshape(1, 1, 1, 2, 1, 1)
    r = np.arange(32).reshape(1, 1, 1, 1, 32, 1)
    c = np.arange(32).reshape(1, 1, 1, 1, 1, 32)
    i = 4 * r + 2 * kh + a
    j = 4 * c + 2 * kw + b
    return (i * 128 + j).reshape(-1).astype(np.int32)


_PERM = _build_perm()


def _sc_gather(table, idx2d):
    info = plsc.get_sparse_core_info()
    nw = info.num_cores * info.num_subcores
    per_w = N_INIT // nw
    nch = per_w // CHUNK
    mesh = plsc.VectorSubcoreMesh(core_axis_name="c", subcore_axis_name="s")

    @functools.partial(
        pl.kernel,
        mesh=mesh,
        out_type=jax.ShapeDtypeStruct((N_INIT, C), jnp.float32),
        scratch_types=[
            pltpu.VMEM((nch, CHUNK), jnp.int32),
            pltpu.VMEM((CHUNK, C), jnp.float32),
            pltpu.VMEM((CHUNK, C), jnp.float32),
            pltpu.SemaphoreType.DMA,
            pltpu.SemaphoreType.DMA,
        ],
    )
    def gk(table_hbm, idx_hbm, out_hbm, idx_v, buf0, buf1, sem0, sem1):
        wid = lax.axis_index("s") * info.num_cores + lax.axis_index("c")
        base = wid * per_w
        pltpu.sync_copy(idx_hbm.at[pl.ds(wid * nch, nch)], idx_v)
        bufs, sems = (buf0, buf1), (sem0, sem1)
        handles = [None, None]
        handles[0] = pltpu.async_copy(table_hbm.at[idx_v.at[0]], buf0, sem0)
        for ci in range(nch):
            t = ci % 2
            if ci + 1 < nch:
                handles[1 - t] = pltpu.async_copy(
                    table_hbm.at[idx_v.at[ci + 1]], bufs[1 - t], sems[1 - t])
            handles[t].wait()
            pltpu.sync_copy(bufs[t],
                            out_hbm.at[pl.ds(base + ci * CHUNK, CHUNK)])

    return gk(table, idx2d)


def _kv_path(g4, w2s, srb, lng, lnb, wkv, augb):
    BCC = 256

    def body(g_ref, w2_ref, srb_ref, lng_ref, lnb_ref, wkv_ref, augb_ref,
             out_ref):
        acc = jnp.broadcast_to(srb_ref[...], (BCC, C)).astype(jnp.float32)
        for k in range(4):
            mk = (g_ref[k, 0] + g_ref[k, 1] + g_ref[k, 2] + g_ref[k, 3])
            acc = acc + jnp.dot(mk.astype(jnp.bfloat16), w2_ref[k],
                                preferred_element_type=jnp.float32)
        mu = jnp.mean(acc, axis=-1, keepdims=True)
        xc = acc - mu
        var = jnp.mean(xc * xc, axis=-1, keepdims=True)
        ln = xc * lax.rsqrt(var + EPS) * lng_ref[...] + lnb_ref[...]
        kv2 = jnp.dot(ln.astype(jnp.bfloat16), wkv_ref[...],
                      preferred_element_type=jnp.float32) + augb_ref[...]
        out_ref[...] = kv2.astype(jnp.bfloat16)

    return pl.pallas_call(
        body,
        grid=(NS // BCC,),
        in_specs=[
            pl.BlockSpec((4, 4, BCC, C), lambda i: (0, 0, i, 0)),
            pl.BlockSpec((4, C, C), lambda i: (0, 0, 0)),
            pl.BlockSpec((1, C), lambda i: (0, 0)),
            pl.BlockSpec((1, C), lambda i: (0, 0)),
            pl.BlockSpec((1, C), lambda i: (0, 0)),
            pl.BlockSpec((C, KC), lambda i: (0, 0)),
            pl.BlockSpec((1, KC), lambda i: (0, 0)),
        ],
        out_specs=pl.BlockSpec((BCC, KC), lambda i: (i, 0)),
        out_shape=jax.ShapeDtypeStruct((NS, KC), jnp.bfloat16),
    )(g4, w2s, srb, lng, lnb, wkv, augb)


def _attention(q_x, wqs, kv, wp, bp, out_prev, bsel):
    BQ = 1024

    def body(qx_ref, wq_ref, kv_ref, wp_ref, bp_ref, *rest):
        out_ref = rest[-1]
        q = jnp.dot(qx_ref[0].astype(jnp.bfloat16), wq_ref[...],
                    preferred_element_type=jnp.float32).astype(jnp.bfloat16)
        outs = []
        for h in range(NH):
            qh = q[:, h * HD:(h + 1) * HD]
            kh = kv_ref[:, h * HD:(h + 1) * HD]
            vaug = kv_ref[:, C + h * VB:C + (h + 1) * VB]
            s = lax.dot_general(qh, kh, (((1,), (1,)), ((), ())),
                                preferred_element_type=jnp.float32)
            m = jnp.max(s, axis=-1, keepdims=True)
            p = jnp.exp(s - m).astype(jnp.bfloat16)
            od = jnp.dot(p, vaug, preferred_element_type=jnp.float32)
            outs.append(od[:, :HD] / od[:, HD:HD + 1])
        acc = jnp.concatenate(outs, axis=-1)
        out_ref[0] = jnp.dot(acc.astype(jnp.bfloat16), wp_ref[...],
                             preferred_element_type=jnp.float32) + bp_ref[...]

    in_specs = [
        pl.BlockSpec((1, BQ, C), lambda i: (bsel, i, 0)),
        pl.BlockSpec((C, C), lambda i: (0, 0)),
        pl.BlockSpec((NS, KC), lambda i: (0, 0)),
        pl.BlockSpec((C, C), lambda i: (0, 0)),
        pl.BlockSpec((1, C), lambda i: (0, 0)),
    ]
    args = [q_x, wqs, kv, wp, bp]
    aliases = {}
    if out_prev is not None:
        in_specs.append(pl.BlockSpec((1, BQ, C), lambda i: (bsel, i, 0)))
        args.append(out_prev)
        aliases = {5: 0}
    return pl.pallas_call(
        body,
        grid=(NQ // BQ,),
        in_specs=in_specs,
        out_specs=pl.BlockSpec((1, BQ, C), lambda i: (bsel, i, 0)),
        out_shape=jax.ShapeDtypeStruct((B, NQ, C), jnp.float32),
        input_output_aliases=aliases,
    )(*args)


def kernel(q_x, kv_x, idx_token, Wq, Wkv, sr_w, sr_b, ln_g, ln_b, Wp, bp):
    idx32 = idx_token.astype(jnp.int32)
    w2s = (jnp.transpose(sr_w, (2, 3, 1, 0)).reshape(4, C, C)
           * INV4).astype(jnp.bfloat16)
    wqs = (Wq * SCALE).astype(jnp.bfloat16)
    wkv16 = jnp.zeros((C, KC), jnp.bfloat16)
    wkv16 = wkv16.at[:, :C].set(Wkv[:, :C].astype(jnp.bfloat16))
    for h in range(NH):
        wkv16 = wkv16.at[:, C + h * VB:C + h * VB + HD].set(
            Wkv[:, C + h * HD:C + (h + 1) * HD].astype(jnp.bfloat16))
    augb = jnp.zeros((1, KC), jnp.float32)
    augb = augb.at[0, C + HD + VB * np.arange(NH)].set(1.0)
    wp16 = Wp.astype(jnp.bfloat16)
    srb = sr_b.reshape(1, C)
    lng = ln_g.reshape(1, C)
    lnb = ln_b.reshape(1, C)
    bp2 = bp.reshape(1, C)
    table = kv_x.reshape(B * NKV, C)
    out = None
    for b in range(B):
        idx2d = (idx32[b, _PERM] + b * NKV).reshape(N_INIT // CHUNK, CHUNK)
        g = _sc_gather(table, idx2d)
        g4 = g.reshape(4, 4, NS, C)
        srb_b = srb if b == 0 else srb + out[0, :1, :] * 0.0
        kv = _kv_path(g4, w2s, srb_b, lng, lnb, wkv16, augb)
        out = _attention(q_x, wqs, kv, wp16, bp2, out, b)
    return out
